# Initial kernel scaffold; baseline (speedup 1.0000x reference)
import jax
import jax.numpy as jnp
from jax.experimental import pallas as pl


def kernel(x, W_ih, W_hh, b_ih, b_hh, W_fc, b_fc):
    raise NotImplementedError("write your pallas kernel here")


# speedup vs baseline: 10.2565x; 10.2565x over previous
import jax
import jax.numpy as jnp
from jax import lax
from jax.experimental import pallas as pl
from jax.experimental.pallas import tpu as pltpu

_ALPHA = 0.3
_BETA = 0.5
_TB = 128


def _fused_kernel(x_ref, wih_ref, whh_ref, bih_ref, bhh_ref, wfc_ref, bfc_ref,
                  out_ref, ema1_ref, ema2_ref, h_ref, xs2_ref, xp_ref):
    tb = pl.program_id(1)
    nb = pl.num_programs(1)

    @pl.when(tb == 0)
    def _init():
        h_ref[...] = jnp.zeros_like(h_ref)

    def ema_body(t, carry):
        prev1, prev2 = carry
        xt = x_ref[t]
        first = (tb * _TB + t) == 0
        cur1 = jnp.where(first, xt, (1.0 - _ALPHA) * xt + _ALPHA * prev1)
        cur2 = jnp.where(first, cur1, (1.0 - _BETA) * cur1 + _BETA * prev2)
        xs2_ref[t] = cur2
        return cur1, cur2

    p1, p2 = lax.fori_loop(0, _TB, ema_body, (ema1_ref[...], ema2_ref[...]))
    ema1_ref[...] = p1
    ema2_ref[...] = p2

    xs2 = xs2_ref[...].reshape(_TB * 64, 64)
    xp_ref[...] = (
        jnp.dot(xs2, wih_ref[...], preferred_element_type=jnp.float32)
        + bih_ref[...]
    )

    def gru_body(t, h):
        gi = xp_ref[pl.ds(t * 64, 64), :]
        gh = jnp.dot(h, whh_ref[...], preferred_element_type=jnp.float32) \
            + bhh_ref[...]
        r = jax.nn.sigmoid(gi[:, 0:128] + gh[:, 0:128])
        z = jax.nn.sigmoid(gi[:, 128:256] + gh[:, 128:256])
        n = jnp.tanh(gi[:, 256:384] + r * gh[:, 256:384])
        return (1.0 - z) * n + z * h

    h = lax.fori_loop(0, _TB, gru_body, h_ref[...])
    h_ref[...] = h

    @pl.when(tb == nb - 1)
    def _final():
        out = jnp.dot(h, wfc_ref[...], preferred_element_type=jnp.float32) \
            + bfc_ref[...]
        st11 = p1[:, 1:3]
        st12 = p2[:, 1:3]
        out = (out - _BETA * st12) / (1.0 - _BETA)
        out = (out - _ALPHA * st11) / (1.0 - _ALPHA)
        out_ref[...] = out


def kernel(x, W_ih, W_hh, b_ih, b_hh, W_fc, b_fc):
    B, T, I = x.shape
    H = W_hh.shape[1]
    C = W_fc.shape[0]
    nb = T // _TB
    xt = jnp.swapaxes(x, 0, 1)

    out = pl.pallas_call(
        _fused_kernel,
        grid=(2, nb),
        in_specs=[
            pl.BlockSpec((_TB, B // 2, I), lambda c, t: (t, c, 0)),
            pl.BlockSpec((I, 3 * H), lambda c, t: (0, 0)),
            pl.BlockSpec((H, 3 * H), lambda c, t: (0, 0)),
            pl.BlockSpec((1, 3 * H), lambda c, t: (0, 0)),
            pl.BlockSpec((1, 3 * H), lambda c, t: (0, 0)),
            pl.BlockSpec((H, C), lambda c, t: (0, 0)),
            pl.BlockSpec((1, C), lambda c, t: (0, 0)),
        ],
        out_specs=pl.BlockSpec((B // 2, C), lambda c, t: (c, 0)),
        out_shape=jax.ShapeDtypeStruct((B, C), jnp.float32),
        scratch_shapes=[
            pltpu.VMEM((B // 2, I), jnp.float32),
            pltpu.VMEM((B // 2, I), jnp.float32),
            pltpu.VMEM((B // 2, H), jnp.float32),
            pltpu.VMEM((_TB, B // 2, I), jnp.float32),
            pltpu.VMEM((_TB * (B // 2), 3 * H), jnp.float32),
        ],
        compiler_params=pltpu.CompilerParams(
            dimension_semantics=("parallel", "arbitrary"),
            vmem_limit_bytes=100 * 1024 * 1024,
        ),
    )(xt, W_ih.T, W_hh.T, b_ih.reshape(1, -1), b_hh.reshape(1, -1),
      W_fc.T, b_fc.reshape(1, -1))

    return out[:, None, :]
